# baseline (speedup 1.0000x reference)
import functools

import jax
import jax.numpy as jnp
from jax import lax
from jax.experimental import pallas as pl
from jax.experimental.pallas import tpu as pltpu

try:
    from jax.experimental.pallas import tpu_sc as plsc
except ImportError:
    plsc = None

_NUM_CODES = 8192
_DIM = 32
_BETA = 0.25
_TOK_BLOCK = 128

_NC = 2
_NS = 16
_NW = _NC * _NS


def _argmin_body(zb_ref, eb_ref, a2_ref, b2_ref, idx_ref):
    i = pl.program_id(0)
    ab = lax.dot_general(zb_ref[...], eb_ref[...],
                         (((1,), (1,)), ((), ())),
                         preferred_element_type=jnp.float32)
    dist = a2_ref[...] - 2.0 * ab + b2_ref[...][None, :]
    m = jnp.min(dist, axis=1, keepdims=True)
    ids = lax.broadcasted_iota(jnp.int32, dist.shape, 1)
    idx = jnp.min(jnp.where(dist == m, ids, _NUM_CODES), axis=1)
    idx_ref[pl.ds(i * _TOK_BLOCK, _TOK_BLOCK)] = idx


def _nearest_codes(z_flat, embedding):
    n_tok = z_flat.shape[0]
    grid = n_tok // _TOK_BLOCK
    a2 = jnp.sum(z_flat ** 2, axis=1, keepdims=True)
    b2 = jnp.sum(embedding ** 2, axis=1)
    return pl.pallas_call(
        _argmin_body,
        grid=(grid,),
        in_specs=[
            pl.BlockSpec((_TOK_BLOCK, _DIM), lambda i: (i, 0)),
            pl.BlockSpec((_NUM_CODES, _DIM), lambda i: (0, 0)),
            pl.BlockSpec((_TOK_BLOCK, 1), lambda i: (i, 0)),
            pl.BlockSpec((_NUM_CODES,), lambda i: (0,)),
        ],
        out_specs=pl.BlockSpec((n_tok,), lambda i: (0,)),
        out_shape=jax.ShapeDtypeStruct((n_tok,), jnp.int32),
    )(z_flat.astype(jnp.bfloat16), embedding.astype(jnp.bfloat16), a2, b2)


def _sc_gather_count(embedding, indices):
    n_tok = indices.shape[0]
    bpw = n_tok // _NW
    mesh = plsc.VectorSubcoreMesh(core_axis_name="c", subcore_axis_name="s")

    @functools.partial(
        pl.kernel,
        mesh=mesh,
        compiler_params=pltpu.CompilerParams(
            needs_layout_passes=False, use_tc_tiling_on_sc=False),
        out_type=(
            jax.ShapeDtypeStruct((n_tok, _DIM), jnp.float32),
            jax.ShapeDtypeStruct((_NW, _NUM_CODES), jnp.float32),
        ),
        scratch_types=[
            pltpu.VMEM((bpw,), jnp.int32),
            pltpu.VMEM((bpw, _DIM), jnp.float32),
            pltpu.VMEM((_NUM_CODES,), jnp.float32),
            pltpu.SemaphoreType.DMA,
        ],
    )
    def k(emb_hbm, idx_hbm, q_hbm, cnt_hbm, idx_v, rows_v, cnt_v, sem):
        wid = lax.axis_index("s") * _NC + lax.axis_index("c")
        base = wid * bpw
        pltpu.sync_copy(idx_hbm.at[pl.ds(base, bpw)], idx_v)
        pltpu.async_copy(emb_hbm.at[idx_v], rows_v, sem).wait()
        pltpu.sync_copy(rows_v, q_hbm.at[pl.ds(base, bpw)])

        def zero_body(j, _):
            cnt_v[pl.ds(j * 16, 16)] = jnp.zeros((16,), jnp.float32)
            return 0

        lax.fori_loop(0, _NUM_CODES // 16, zero_body, 0)
        ones = jnp.ones((16,), jnp.float32)
        for j in range(bpw // 16):
            idx16 = idx_v[pl.ds(j * 16, 16)]
            plsc.addupdate_scatter(cnt_v, [idx16], ones)
        pltpu.sync_copy(cnt_v, cnt_hbm.at[wid])

    return k(embedding, indices)


def _final_body(z_ref, q_ref, c_ref, zq_ref, com_ref, per_ref, use_ref):
    z = z_ref[...]
    q = q_ref[...]
    zq_ref[...] = z + (q - z)
    diff = z - q
    n = z.shape[0] * z.shape[1]
    com_ref[0, 0] = _BETA * (jnp.sum(diff * diff) / n)
    counts = jnp.sum(c_ref[...], axis=0)
    avg = counts / float(z.shape[0])
    per_ref[0, 0] = jnp.exp(-jnp.sum(avg * jnp.log(avg + 1e-10)))
    use_ref[0, 0] = jnp.sum((counts > 0).astype(jnp.float32)) / _NUM_CODES


def _finalize(z_flat, q_flat, cnt):
    n_tok = z_flat.shape[0]
    scalar = jax.ShapeDtypeStruct((1, 1), jnp.float32)
    return pl.pallas_call(
        _final_body,
        out_shape=(
            jax.ShapeDtypeStruct((n_tok, _DIM), jnp.float32),
            scalar, scalar, scalar,
        ),
        out_specs=(
            pl.BlockSpec(memory_space=pltpu.VMEM),
            pl.BlockSpec(memory_space=pltpu.SMEM),
            pl.BlockSpec(memory_space=pltpu.SMEM),
            pl.BlockSpec(memory_space=pltpu.SMEM),
        ),
    )(z_flat, q_flat, cnt)


def kernel(z_e, embedding):
    b, d, h, w = z_e.shape
    z_flat = jnp.transpose(z_e, (0, 2, 3, 1)).reshape(-1, d)
    indices = _nearest_codes(z_flat, embedding)
    q_flat, cnt = _sc_gather_count(embedding, indices)
    zq_flat, com, per, use = _finalize(z_flat, q_flat, cnt)
    z_q = jnp.transpose(zq_flat.reshape(b, h, w, d), (0, 3, 1, 2))
    indices_out = indices.reshape(b, h, w)
    commit_loss = com.reshape(())
    codebook_loss = jnp.zeros(())
    perplexity = per.reshape(())
    usage = use.reshape(())
    return (z_q, indices_out, commit_loss, codebook_loss, perplexity, usage)


# speedup vs baseline: 1.1210x; 1.1210x over previous
import functools

import jax
import jax.numpy as jnp
from jax import lax
from jax.experimental import pallas as pl
from jax.experimental.pallas import tpu as pltpu

try:
    from jax.experimental.pallas import tpu_sc as plsc
except ImportError:
    plsc = None

_NUM_CODES = 8192
_DIM = 32
_BETA = 0.25
_TOK_BLOCK = 256

_NC = 2
_NS = 16
_NW = _NC * _NS


def _argmin_body(zb_ref, eb_ref, a2_ref, b2_ref, idx_ref):
    i = pl.program_id(0)
    ab = lax.dot_general(zb_ref[...], eb_ref[...],
                         (((1,), (1,)), ((), ())),
                         preferred_element_type=jnp.float32)
    dist = a2_ref[...] - 2.0 * ab + b2_ref[...][None, :]
    m = jnp.min(dist, axis=1, keepdims=True)
    ids = lax.broadcasted_iota(jnp.int32, dist.shape, 1)
    idx = jnp.min(jnp.where(dist == m, ids, _NUM_CODES), axis=1)
    idx_ref[pl.ds(i * _TOK_BLOCK, _TOK_BLOCK)] = idx


def _nearest_codes(z_flat, embedding):
    n_tok = z_flat.shape[0]
    grid = n_tok // _TOK_BLOCK
    a2 = jnp.sum(z_flat ** 2, axis=1, keepdims=True)
    b2 = jnp.sum(embedding ** 2, axis=1)
    return pl.pallas_call(
        _argmin_body,
        grid=(grid,),
        in_specs=[
            pl.BlockSpec((_TOK_BLOCK, _DIM), lambda i: (i, 0)),
            pl.BlockSpec((_NUM_CODES, _DIM), lambda i: (0, 0)),
            pl.BlockSpec((_TOK_BLOCK, 1), lambda i: (i, 0)),
            pl.BlockSpec((_NUM_CODES,), lambda i: (0,)),
        ],
        out_specs=pl.BlockSpec((n_tok,), lambda i: (0,)),
        out_shape=jax.ShapeDtypeStruct((n_tok,), jnp.int32),
    )(z_flat.astype(jnp.bfloat16), embedding.astype(jnp.bfloat16), a2, b2)


def _sc_gather_count(embedding, indices):
    n_tok = indices.shape[0]
    bpw = n_tok // _NW
    mesh = plsc.VectorSubcoreMesh(core_axis_name="c", subcore_axis_name="s")

    @functools.partial(
        pl.kernel,
        mesh=mesh,
        compiler_params=pltpu.CompilerParams(
            needs_layout_passes=False, use_tc_tiling_on_sc=False),
        out_type=(
            jax.ShapeDtypeStruct((n_tok, _DIM), jnp.float32),
            jax.ShapeDtypeStruct((_NW, _NUM_CODES), jnp.float32),
        ),
        scratch_types=[
            pltpu.VMEM((bpw,), jnp.int32),
            pltpu.VMEM((bpw, _DIM), jnp.float32),
            pltpu.VMEM((_NUM_CODES,), jnp.float32),
            pltpu.SemaphoreType.DMA,
        ],
    )
    def k(emb_hbm, idx_hbm, q_hbm, cnt_hbm, idx_v, rows_v, cnt_v, sem):
        wid = lax.axis_index("s") * _NC + lax.axis_index("c")
        base = wid * bpw
        pltpu.sync_copy(idx_hbm.at[pl.ds(base, bpw)], idx_v)
        pltpu.async_copy(emb_hbm.at[idx_v], rows_v, sem).wait()
        pltpu.sync_copy(rows_v, q_hbm.at[pl.ds(base, bpw)])

        def zero_body(j, _):
            cnt_v[pl.ds(j * 16, 16)] = jnp.zeros((16,), jnp.float32)
            return 0

        lax.fori_loop(0, _NUM_CODES // 16, zero_body, 0)
        ones = jnp.ones((16,), jnp.float32)
        for j in range(bpw // 16):
            idx16 = idx_v[pl.ds(j * 16, 16)]
            plsc.addupdate_scatter(cnt_v, [idx16], ones)
        pltpu.sync_copy(cnt_v, cnt_hbm.at[wid])

    return k(embedding, indices)


def _final_body(z_ref, q_ref, c_ref, zq_ref, com_ref, per_ref, use_ref):
    z = z_ref[...]
    q = q_ref[...]
    zq_ref[...] = z + (q - z)
    diff = z - q
    n = z.shape[0] * z.shape[1]
    com_ref[0, 0] = _BETA * (jnp.sum(diff * diff) / n)
    counts = jnp.sum(c_ref[...], axis=0)
    avg = counts / float(z.shape[0])
    per_ref[0, 0] = jnp.exp(-jnp.sum(avg * jnp.log(avg + 1e-10)))
    use_ref[0, 0] = jnp.sum((counts > 0).astype(jnp.float32)) / _NUM_CODES


def _finalize(z_flat, q_flat, cnt):
    n_tok = z_flat.shape[0]
    scalar = jax.ShapeDtypeStruct((1, 1), jnp.float32)
    return pl.pallas_call(
        _final_body,
        out_shape=(
            jax.ShapeDtypeStruct((n_tok, _DIM), jnp.float32),
            scalar, scalar, scalar,
        ),
        out_specs=(
            pl.BlockSpec(memory_space=pltpu.VMEM),
            pl.BlockSpec(memory_space=pltpu.SMEM),
            pl.BlockSpec(memory_space=pltpu.SMEM),
            pl.BlockSpec(memory_space=pltpu.SMEM),
        ),
    )(z_flat, q_flat, cnt)


def kernel(z_e, embedding):
    b, d, h, w = z_e.shape
    z_flat = jnp.transpose(z_e, (0, 2, 3, 1)).reshape(-1, d)
    indices = _nearest_codes(z_flat, embedding)
    q_flat, cnt = _sc_gather_count(embedding, indices)
    zq_flat, com, per, use = _finalize(z_flat, q_flat, cnt)
    z_q = jnp.transpose(zq_flat.reshape(b, h, w, d), (0, 3, 1, 2))
    indices_out = indices.reshape(b, h, w)
    commit_loss = com.reshape(())
    codebook_loss = jnp.zeros(())
    perplexity = per.reshape(())
    usage = use.reshape(())
    return (z_q, indices_out, commit_loss, codebook_loss, perplexity, usage)


# speedup vs baseline: 1.1504x; 1.0262x over previous
import functools

import jax
import jax.numpy as jnp
from jax import lax
from jax.experimental import pallas as pl
from jax.experimental.pallas import tpu as pltpu

try:
    from jax.experimental.pallas import tpu_sc as plsc
except ImportError:
    plsc = None

_NUM_CODES = 8192
_DIM = 32
_BETA = 0.25
_TOK_BLOCK = 512

_NC = 2
_NS = 16
_NW = _NC * _NS


def _argmin_body(zb_ref, eb_ref, a2_ref, b2_ref, idx_ref):
    i = pl.program_id(0)
    ab = lax.dot_general(zb_ref[...], eb_ref[...],
                         (((1,), (1,)), ((), ())),
                         preferred_element_type=jnp.float32)
    dist = a2_ref[...] - 2.0 * ab + b2_ref[...][None, :]
    m = jnp.min(dist, axis=1, keepdims=True)
    ids = lax.broadcasted_iota(jnp.int32, dist.shape, 1)
    idx = jnp.min(jnp.where(dist == m, ids, _NUM_CODES), axis=1)
    idx_ref[pl.ds(i * _TOK_BLOCK, _TOK_BLOCK)] = idx


def _nearest_codes(z_flat, embedding):
    n_tok = z_flat.shape[0]
    grid = n_tok // _TOK_BLOCK
    a2 = jnp.sum(z_flat ** 2, axis=1, keepdims=True)
    b2 = jnp.sum(embedding ** 2, axis=1)
    return pl.pallas_call(
        _argmin_body,
        grid=(grid,),
        in_specs=[
            pl.BlockSpec((_TOK_BLOCK, _DIM), lambda i: (i, 0)),
            pl.BlockSpec((_NUM_CODES, _DIM), lambda i: (0, 0)),
            pl.BlockSpec((_TOK_BLOCK, 1), lambda i: (i, 0)),
            pl.BlockSpec((_NUM_CODES,), lambda i: (0,)),
        ],
        out_specs=pl.BlockSpec((n_tok,), lambda i: (0,)),
        out_shape=jax.ShapeDtypeStruct((n_tok,), jnp.int32),
    )(z_flat.astype(jnp.bfloat16), embedding.astype(jnp.bfloat16), a2, b2)


def _sc_gather_count(embedding, indices):
    n_tok = indices.shape[0]
    bpw = n_tok // _NW
    mesh = plsc.VectorSubcoreMesh(core_axis_name="c", subcore_axis_name="s")

    @functools.partial(
        pl.kernel,
        mesh=mesh,
        compiler_params=pltpu.CompilerParams(
            needs_layout_passes=False, use_tc_tiling_on_sc=False),
        out_type=(
            jax.ShapeDtypeStruct((n_tok, _DIM), jnp.float32),
            jax.ShapeDtypeStruct((_NW, _NUM_CODES), jnp.float32),
        ),
        scratch_types=[
            pltpu.VMEM((bpw,), jnp.int32),
            pltpu.VMEM((bpw, _DIM), jnp.float32),
            pltpu.VMEM((_NUM_CODES,), jnp.float32),
            pltpu.SemaphoreType.DMA,
        ],
    )
    def k(emb_hbm, idx_hbm, q_hbm, cnt_hbm, idx_v, rows_v, cnt_v, sem):
        wid = lax.axis_index("s") * _NC + lax.axis_index("c")
        base = wid * bpw
        pltpu.sync_copy(idx_hbm.at[pl.ds(base, bpw)], idx_v)
        pltpu.async_copy(emb_hbm.at[idx_v], rows_v, sem).wait()
        pltpu.sync_copy(rows_v, q_hbm.at[pl.ds(base, bpw)])

        def zero_body(j, _):
            cnt_v[pl.ds(j * 16, 16)] = jnp.zeros((16,), jnp.float32)
            return 0

        lax.fori_loop(0, _NUM_CODES // 16, zero_body, 0)
        ones = jnp.ones((16,), jnp.float32)
        for j in range(bpw // 16):
            idx16 = idx_v[pl.ds(j * 16, 16)]
            plsc.addupdate_scatter(cnt_v, [idx16], ones)
        pltpu.sync_copy(cnt_v, cnt_hbm.at[wid])

    return k(embedding, indices)


def _final_body(z_ref, q_ref, c_ref, zq_ref, com_ref, per_ref, use_ref):
    z = z_ref[...]
    q = q_ref[...]
    zq_ref[...] = z + (q - z)
    diff = z - q
    n = z.shape[0] * z.shape[1]
    com_ref[0, 0] = _BETA * (jnp.sum(diff * diff) / n)
    counts = jnp.sum(c_ref[...], axis=0)
    avg = counts / float(z.shape[0])
    per_ref[0, 0] = jnp.exp(-jnp.sum(avg * jnp.log(avg + 1e-10)))
    use_ref[0, 0] = jnp.sum((counts > 0).astype(jnp.float32)) / _NUM_CODES


def _finalize(z_flat, q_flat, cnt):
    n_tok = z_flat.shape[0]
    scalar = jax.ShapeDtypeStruct((1, 1), jnp.float32)
    return pl.pallas_call(
        _final_body,
        out_shape=(
            jax.ShapeDtypeStruct((n_tok, _DIM), jnp.float32),
            scalar, scalar, scalar,
        ),
        out_specs=(
            pl.BlockSpec(memory_space=pltpu.VMEM),
            pl.BlockSpec(memory_space=pltpu.SMEM),
            pl.BlockSpec(memory_space=pltpu.SMEM),
            pl.BlockSpec(memory_space=pltpu.SMEM),
        ),
    )(z_flat, q_flat, cnt)


def kernel(z_e, embedding):
    b, d, h, w = z_e.shape
    z_flat = jnp.transpose(z_e, (0, 2, 3, 1)).reshape(-1, d)
    indices = _nearest_codes(z_flat, embedding)
    q_flat, cnt = _sc_gather_count(embedding, indices)
    zq_flat, com, per, use = _finalize(z_flat, q_flat, cnt)
    z_q = jnp.transpose(zq_flat.reshape(b, h, w, d), (0, 3, 1, 2))
    indices_out = indices.reshape(b, h, w)
    commit_loss = com.reshape(())
    codebook_loss = jnp.zeros(())
    perplexity = per.reshape(())
    usage = use.reshape(())
    return (z_q, indices_out, commit_loss, codebook_loss, perplexity, usage)


# speedup vs baseline: 1.1545x; 1.0036x over previous
import functools

import jax
import jax.numpy as jnp
from jax import lax
from jax.experimental import pallas as pl
from jax.experimental.pallas import tpu as pltpu

try:
    from jax.experimental.pallas import tpu_sc as plsc
except ImportError:
    plsc = None

_NUM_CODES = 8192
_DIM = 32
_BETA = 0.25
_TOK_BLOCK = 1024

_NC = 2
_NS = 16
_NW = _NC * _NS


def _argmin_body(zb_ref, eb_ref, a2_ref, b2_ref, idx_ref):
    i = pl.program_id(0)
    ab = lax.dot_general(zb_ref[...], eb_ref[...],
                         (((1,), (1,)), ((), ())),
                         preferred_element_type=jnp.float32)
    dist = a2_ref[...] - 2.0 * ab + b2_ref[...][None, :]
    m = jnp.min(dist, axis=1, keepdims=True)
    ids = lax.broadcasted_iota(jnp.int32, dist.shape, 1)
    idx = jnp.min(jnp.where(dist == m, ids, _NUM_CODES), axis=1)
    idx_ref[pl.ds(i * _TOK_BLOCK, _TOK_BLOCK)] = idx


def _nearest_codes(z_flat, embedding):
    n_tok = z_flat.shape[0]
    grid = n_tok // _TOK_BLOCK
    a2 = jnp.sum(z_flat ** 2, axis=1, keepdims=True)
    b2 = jnp.sum(embedding ** 2, axis=1)
    return pl.pallas_call(
        _argmin_body,
        grid=(grid,),
        in_specs=[
            pl.BlockSpec((_TOK_BLOCK, _DIM), lambda i: (i, 0)),
            pl.BlockSpec((_NUM_CODES, _DIM), lambda i: (0, 0)),
            pl.BlockSpec((_TOK_BLOCK, 1), lambda i: (i, 0)),
            pl.BlockSpec((_NUM_CODES,), lambda i: (0,)),
        ],
        out_specs=pl.BlockSpec((n_tok,), lambda i: (0,)),
        out_shape=jax.ShapeDtypeStruct((n_tok,), jnp.int32),
    )(z_flat.astype(jnp.bfloat16), embedding.astype(jnp.bfloat16), a2, b2)


def _sc_gather_count(embedding, indices):
    n_tok = indices.shape[0]
    bpw = n_tok // _NW
    mesh = plsc.VectorSubcoreMesh(core_axis_name="c", subcore_axis_name="s")

    @functools.partial(
        pl.kernel,
        mesh=mesh,
        compiler_params=pltpu.CompilerParams(
            needs_layout_passes=False, use_tc_tiling_on_sc=False),
        out_type=(
            jax.ShapeDtypeStruct((n_tok, _DIM), jnp.float32),
            jax.ShapeDtypeStruct((_NW, _NUM_CODES), jnp.float32),
        ),
        scratch_types=[
            pltpu.VMEM((bpw,), jnp.int32),
            pltpu.VMEM((bpw, _DIM), jnp.float32),
            pltpu.VMEM((_NUM_CODES,), jnp.float32),
            pltpu.SemaphoreType.DMA,
        ],
    )
    def k(emb_hbm, idx_hbm, q_hbm, cnt_hbm, idx_v, rows_v, cnt_v, sem):
        wid = lax.axis_index("s") * _NC + lax.axis_index("c")
        base = wid * bpw
        pltpu.sync_copy(idx_hbm.at[pl.ds(base, bpw)], idx_v)
        pltpu.async_copy(emb_hbm.at[idx_v], rows_v, sem).wait()
        pltpu.sync_copy(rows_v, q_hbm.at[pl.ds(base, bpw)])

        def zero_body(j, _):
            cnt_v[pl.ds(j * 16, 16)] = jnp.zeros((16,), jnp.float32)
            return 0

        lax.fori_loop(0, _NUM_CODES // 16, zero_body, 0)
        ones = jnp.ones((16,), jnp.float32)
        for j in range(bpw // 16):
            idx16 = idx_v[pl.ds(j * 16, 16)]
            plsc.addupdate_scatter(cnt_v, [idx16], ones)
        pltpu.sync_copy(cnt_v, cnt_hbm.at[wid])

    return k(embedding, indices)


def _final_body(z_ref, q_ref, c_ref, zq_ref, com_ref, per_ref, use_ref):
    z = z_ref[...]
    q = q_ref[...]
    zq_ref[...] = z + (q - z)
    diff = z - q
    n = z.shape[0] * z.shape[1]
    com_ref[0, 0] = _BETA * (jnp.sum(diff * diff) / n)
    counts = jnp.sum(c_ref[...], axis=0)
    avg = counts / float(z.shape[0])
    per_ref[0, 0] = jnp.exp(-jnp.sum(avg * jnp.log(avg + 1e-10)))
    use_ref[0, 0] = jnp.sum((counts > 0).astype(jnp.float32)) / _NUM_CODES


def _finalize(z_flat, q_flat, cnt):
    n_tok = z_flat.shape[0]
    scalar = jax.ShapeDtypeStruct((1, 1), jnp.float32)
    return pl.pallas_call(
        _final_body,
        out_shape=(
            jax.ShapeDtypeStruct((n_tok, _DIM), jnp.float32),
            scalar, scalar, scalar,
        ),
        out_specs=(
            pl.BlockSpec(memory_space=pltpu.VMEM),
            pl.BlockSpec(memory_space=pltpu.SMEM),
            pl.BlockSpec(memory_space=pltpu.SMEM),
            pl.BlockSpec(memory_space=pltpu.SMEM),
        ),
    )(z_flat, q_flat, cnt)


def kernel(z_e, embedding):
    b, d, h, w = z_e.shape
    z_flat = jnp.transpose(z_e, (0, 2, 3, 1)).reshape(-1, d)
    indices = _nearest_codes(z_flat, embedding)
    q_flat, cnt = _sc_gather_count(embedding, indices)
    zq_flat, com, per, use = _finalize(z_flat, q_flat, cnt)
    z_q = jnp.transpose(zq_flat.reshape(b, h, w, d), (0, 3, 1, 2))
    indices_out = indices.reshape(b, h, w)
    commit_loss = com.reshape(())
    codebook_loss = jnp.zeros(())
    perplexity = per.reshape(())
    usage = use.reshape(())
    return (z_q, indices_out, commit_loss, codebook_loss, perplexity, usage)
